# Initial kernel scaffold; baseline (speedup 1.0000x reference)
import jax
import jax.numpy as jnp
from jax.experimental import pallas as pl


def kernel(user_emb, item_emb, edge_index, edge_weight):
    raise NotImplementedError("write your pallas kernel here")


# speedup vs baseline: 2.0421x; 2.0421x over previous
import functools

import jax
import jax.numpy as jnp
from jax import lax
from jax.experimental import pallas as pl
from jax.experimental.pallas import tpu as pltpu
from jax.experimental.pallas import tpu_sc as plsc

N_CORES = 2
N_SUBCORES = 16
N_WORKERS = N_CORES * N_SUBCORES
CHUNK = 128
GROUP = 8
GCN_LAYERS_K = 3


@functools.partial(jax.jit, static_argnames=("n", "half_d", "n_groups", "rows_per_tile", "sub_rows"))
def _gcn_call(table0, src_both, dst_r, w_r, *, n, half_d, n_groups, rows_per_tile, sub_rows):
    n_sub = rows_per_tile // sub_rows
    mesh = plsc.VectorSubcoreMesh(core_axis_name="c", subcore_axis_name="s")

    @functools.partial(
        pl.kernel,
        mesh=mesh,
        compiler_params=pltpu.CompilerParams(use_tc_tiling_on_sc=False),
        out_type=[
            jax.ShapeDtypeStruct((2 * n, half_d), jnp.float32),
            jax.ShapeDtypeStruct((2 * n, half_d), jnp.float32),
        ],
        scratch_types=[
            pltpu.VMEM_SHARED((n, half_d), jnp.float32),
            pltpu.VMEM_SHARED((n, half_d), jnp.float32),
            pltpu.VMEM((GROUP, CHUNK), jnp.int32),
            pltpu.VMEM((GROUP, CHUNK), jnp.int32),
            pltpu.VMEM((GROUP, CHUNK), jnp.float32),
            pltpu.VMEM((CHUNK, half_d), jnp.float32),
            pltpu.VMEM((sub_rows, half_d), jnp.float32),
            pltpu.VMEM((sub_rows, half_d), jnp.float32),
            pltpu.VMEM((sub_rows, half_d), jnp.float32),
        ],
    )
    def gcn(table_hbm, src_hbm, dst_hbm, w_hbm, out_hbm, tmp_hbm,
            acc_sh, sum_sh, src_v, dst_v, w_v, rows_v, stA, stB, zer):
        c = lax.axis_index("c")
        s = lax.axis_index("s")
        wid = c * N_SUBCORES + s
        r0 = s * rows_per_tile
        hbm0 = c * n

        def zero_body(i, _):
            for j in range(half_d // 16):
                zer[i, pl.ds(j * 16, 16)] = jnp.zeros((16,), jnp.float32)
            return 0
        lax.fori_loop(0, sub_rows, zero_body, 0)
        for sub in range(n_sub):
            rows = pl.ds(r0 + sub * sub_rows, sub_rows)
            pltpu.sync_copy(zer, acc_sh.at[rows])
            pltpu.sync_copy(table_hbm.at[pl.ds(hbm0 + r0 + sub * sub_rows, sub_rows)], stA)
            pltpu.sync_copy(stA, sum_sh.at[rows])
        plsc.subcore_barrier()

        def do_layer(gather_ref, is_last):
            def group_body(grp, _):
                g0 = grp * GROUP
                pltpu.sync_copy(src_hbm.at[wid, pl.ds(g0, GROUP)], src_v)
                pltpu.sync_copy(dst_hbm.at[s, pl.ds(g0, GROUP)], dst_v)
                pltpu.sync_copy(w_hbm.at[s, pl.ds(g0, GROUP)], w_v)
                for g in range(GROUP):
                    pltpu.sync_copy(gather_ref.at[src_v.at[g]], rows_v)

                    def edge16(kk, _):
                        wvec = w_v[g, pl.ds(kk * 16, 16)]
                        for e in range(16):
                            i = kk * 16 + e
                            wv = wvec[e]
                            for j in range(half_d // 16):
                                sl = pl.ds(j * 16, 16)
                                rows_v[i, sl] = rows_v[i, sl] * wv
                        return 0
                    lax.fori_loop(0, CHUNK // 16, edge16, 0)

                    pltpu.sync_copy(rows_v, acc_sh.at[dst_v.at[g]], add=True)
                return 0
            lax.fori_loop(0, n_groups, group_body, 0)
            plsc.subcore_barrier()

            for sub in range(n_sub):
                rows = pl.ds(r0 + sub * sub_rows, sub_rows)
                hrows = pl.ds(hbm0 + r0 + sub * sub_rows, sub_rows)
                pltpu.sync_copy(acc_sh.at[rows], stA)
                pltpu.sync_copy(sum_sh.at[rows], stB)

                def add_body(i, _):
                    for j in range(half_d // 16):
                        sl = pl.ds(j * 16, 16)
                        v = stB[i, sl] + stA[i, sl]
                        if is_last:
                            v = v * (1.0 / (GCN_LAYERS_K + 1))
                        stB[i, sl] = v
                    return 0
                lax.fori_loop(0, sub_rows, add_body, 0)

                if is_last:
                    pltpu.sync_copy(stB, out_hbm.at[hrows])
                else:
                    pltpu.sync_copy(stB, sum_sh.at[rows])
                    pltpu.sync_copy(stA, tmp_hbm.at[hrows])
                    pltpu.sync_copy(zer, acc_sh.at[rows])
            plsc.subcore_barrier()

        do_layer(table_hbm, False)
        for layer in range(1, GCN_LAYERS_K):
            do_layer(tmp_hbm, layer == GCN_LAYERS_K - 1)

    return gcn(table0, src_both, dst_r, w_r)


def kernel(user_emb, item_emb, edge_index, edge_weight):
    u, d = user_emb.shape
    n = u + item_emb.shape[0]
    e = edge_weight.shape[0]
    half_d = d // 2

    sub_rows = 128
    n_pad = -(-n // (N_SUBCORES * sub_rows)) * (N_SUBCORES * sub_rows)
    rows_per_tile = n_pad // N_SUBCORES

    ept = -(-e // N_SUBCORES)
    ept = -(-ept // (GROUP * CHUNK)) * (GROUP * CHUNK)
    n_chunks = ept // CHUNK
    n_groups = n_chunks // GROUP
    e_pad = ept * N_SUBCORES

    all_emb = jnp.concatenate([user_emb, item_emb], axis=0)
    all_emb = jnp.pad(all_emb, ((0, n_pad - n), (0, 0)))
    table0 = jnp.concatenate([all_emb[:, :half_d], all_emb[:, half_d:]], axis=0)

    src = edge_index[0].astype(jnp.int32)
    dst = edge_index[1].astype(jnp.int32)
    w = edge_weight.astype(jnp.float32)
    pad = e_pad - e
    if pad:
        src = jnp.pad(src, (0, pad))
        dst = jnp.pad(dst, (0, pad))
        w = jnp.pad(w, (0, pad))

    src_both = jnp.stack([src, src + n_pad]).reshape(N_WORKERS, n_chunks, CHUNK)
    dst_r = dst.reshape(N_SUBCORES, n_chunks, CHUNK)
    w_r = w.reshape(N_SUBCORES, n_chunks, CHUNK)

    out, _ = _gcn_call(table0, src_both, dst_r, w_r, n=n_pad, half_d=half_d,
                       n_groups=n_groups, rows_per_tile=rows_per_tile,
                       sub_rows=sub_rows)
    final = jnp.concatenate([out[:n], out[n_pad:n_pad + n]], axis=1)
    return final[:u], final[u:]


# speedup vs baseline: 2.9112x; 1.4256x over previous
import functools

import jax
import jax.numpy as jnp
from jax import lax
from jax.experimental import pallas as pl
from jax.experimental.pallas import tpu as pltpu
from jax.experimental.pallas import tpu_sc as plsc

N_CORES = 2
N_SUBCORES = 16
N_WORKERS = N_CORES * N_SUBCORES
CHUNK = 128
GROUP = 8
ZROWS = 64
GCN_LAYERS_K = 3


@functools.partial(jax.jit, static_argnames=("n", "half_d", "n_groups", "rows_per_tile", "sub_rows"))
def _gcn_call(table0, src_r, dst_r, w_r, *, n, half_d, n_groups, rows_per_tile, sub_rows):
    n_sub = rows_per_tile // sub_rows
    mesh = plsc.VectorSubcoreMesh(core_axis_name="c", subcore_axis_name="s")

    @functools.partial(
        pl.kernel,
        mesh=mesh,
        compiler_params=pltpu.CompilerParams(use_tc_tiling_on_sc=False),
        out_type=jax.ShapeDtypeStruct((2 * n, half_d), jnp.float32),
        scratch_types=[
            pltpu.VMEM_SHARED((n, half_d), jnp.float32),
            pltpu.VMEM_SHARED((n, half_d), jnp.float32),
            pltpu.VMEM((GROUP, CHUNK), jnp.int32),
            pltpu.VMEM((GROUP, CHUNK), jnp.int32),
            pltpu.VMEM((GROUP, CHUNK), jnp.float32),
            pltpu.VMEM((CHUNK, half_d), jnp.float32),
            pltpu.VMEM((sub_rows, half_d), jnp.float32),
            pltpu.VMEM((sub_rows, half_d), jnp.float32),
            pltpu.VMEM((ZROWS, half_d), jnp.float32),
        ],
    )
    def gcn(table_hbm, src_hbm, dst_hbm, w_hbm, out_hbm,
            ping_sh, pong_sh, src_v, dst_v, w_v, rows_v, stA, stB, zer):
        c = lax.axis_index("c")
        s = lax.axis_index("s")
        r0 = s * rows_per_tile
        hbm0 = c * n

        def zero_body(i, _):
            for j in range(half_d // 16):
                zer[i, pl.ds(j * 16, 16)] = jnp.zeros((16,), jnp.float32)
            return 0
        lax.fori_loop(0, ZROWS, zero_body, 0)

        for sub in range(n_sub):
            rows = pl.ds(r0 + sub * sub_rows, sub_rows)
            hrows = pl.ds(hbm0 + r0 + sub * sub_rows, sub_rows)
            pltpu.sync_copy(table_hbm.at[hrows], stA)
            pltpu.sync_copy(stA, ping_sh.at[rows])
            pltpu.sync_copy(stA, out_hbm.at[hrows])
        for z in range(rows_per_tile // ZROWS):
            pltpu.sync_copy(zer, pong_sh.at[pl.ds(r0 + z * ZROWS, ZROWS)])
        plsc.subcore_barrier()

        def do_layer(gather_sh, acc_sh, is_last):
            def group_body(grp, _):
                g0 = grp * GROUP
                pltpu.sync_copy(src_hbm.at[s, pl.ds(g0, GROUP)], src_v)
                pltpu.sync_copy(dst_hbm.at[s, pl.ds(g0, GROUP)], dst_v)
                pltpu.sync_copy(w_hbm.at[s, pl.ds(g0, GROUP)], w_v)
                for g in range(GROUP):
                    pltpu.sync_copy(gather_sh.at[src_v.at[g]], rows_v)

                    def edge16(kk, _):
                        wvec = w_v[g, pl.ds(kk * 16, 16)]
                        for e in range(16):
                            i = kk * 16 + e
                            wv = wvec[e]
                            for j in range(half_d // 16):
                                sl = pl.ds(j * 16, 16)
                                rows_v[i, sl] = rows_v[i, sl] * wv
                        return 0
                    lax.fori_loop(0, CHUNK // 16, edge16, 0)

                    pltpu.sync_copy(rows_v, acc_sh.at[dst_v.at[g]], add=True)
                return 0
            lax.fori_loop(0, n_groups, group_body, 0)
            plsc.subcore_barrier()

            for sub in range(n_sub):
                rows = pl.ds(r0 + sub * sub_rows, sub_rows)
                hrows = pl.ds(hbm0 + r0 + sub * sub_rows, sub_rows)
                pltpu.sync_copy(acc_sh.at[rows], stA)
                pltpu.sync_copy(out_hbm.at[hrows], stB)

                def add_body(i, _):
                    for j in range(half_d // 16):
                        sl = pl.ds(j * 16, 16)
                        v = stB[i, sl] + stA[i, sl]
                        if is_last:
                            v = v * (1.0 / (GCN_LAYERS_K + 1))
                        stB[i, sl] = v
                    return 0
                lax.fori_loop(0, sub_rows, add_body, 0)
                pltpu.sync_copy(stB, out_hbm.at[hrows])

            if not is_last:
                for z in range(rows_per_tile // ZROWS):
                    pltpu.sync_copy(zer, gather_sh.at[pl.ds(r0 + z * ZROWS, ZROWS)])
            plsc.subcore_barrier()

        do_layer(ping_sh, pong_sh, False)
        do_layer(pong_sh, ping_sh, False)
        do_layer(ping_sh, pong_sh, True)

    return gcn(table0, src_r, dst_r, w_r)


def kernel(user_emb, item_emb, edge_index, edge_weight):
    u, d = user_emb.shape
    n = u + item_emb.shape[0]
    e = edge_weight.shape[0]
    half_d = d // 2

    sub_rows = 128
    n_pad = -(-n // (N_SUBCORES * sub_rows)) * (N_SUBCORES * sub_rows)
    rows_per_tile = n_pad // N_SUBCORES

    ept = -(-e // N_SUBCORES)
    ept = -(-ept // (GROUP * CHUNK)) * (GROUP * CHUNK)
    n_chunks = ept // CHUNK
    n_groups = n_chunks // GROUP
    e_pad = ept * N_SUBCORES

    all_emb = jnp.concatenate([user_emb, item_emb], axis=0)
    all_emb = jnp.pad(all_emb, ((0, n_pad - n), (0, 0)))
    table0 = jnp.concatenate([all_emb[:, :half_d], all_emb[:, half_d:]], axis=0)

    src = edge_index[0].astype(jnp.int32)
    dst = edge_index[1].astype(jnp.int32)
    w = edge_weight.astype(jnp.float32)
    pad = e_pad - e
    if pad:
        src = jnp.pad(src, (0, pad))
        dst = jnp.pad(dst, (0, pad))
        w = jnp.pad(w, (0, pad))

    src_r = src.reshape(N_SUBCORES, n_chunks, CHUNK)
    dst_r = dst.reshape(N_SUBCORES, n_chunks, CHUNK)
    w_r = w.reshape(N_SUBCORES, n_chunks, CHUNK)

    out = _gcn_call(table0, src_r, dst_r, w_r, n=n_pad, half_d=half_d,
                    n_groups=n_groups, rows_per_tile=rows_per_tile,
                    sub_rows=sub_rows)
    final = jnp.concatenate([out[:n], out[n_pad:n_pad + n]], axis=1)
    return final[:u], final[u:]


# speedup vs baseline: 3.6672x; 1.2597x over previous
import functools

import jax
import jax.numpy as jnp
from jax import lax
from jax.experimental import pallas as pl
from jax.experimental.pallas import tpu as pltpu
from jax.experimental.pallas import tpu_sc as plsc

N_CORES = 2
N_SUBCORES = 16
N_WORKERS = N_CORES * N_SUBCORES
CHUNK = 128
GROUP = 8
ZROWS = 64
GCN_LAYERS_K = 3


@functools.partial(jax.jit, static_argnames=("n", "half_d", "n_groups", "rows_per_tile", "sub_rows"))
def _gcn_call(table0, src_r, dst_r, w_r, *, n, half_d, n_groups, rows_per_tile, sub_rows):
    n_sub = rows_per_tile // sub_rows
    mesh = plsc.VectorSubcoreMesh(core_axis_name="c", subcore_axis_name="s")

    @functools.partial(
        pl.kernel,
        mesh=mesh,
        compiler_params=pltpu.CompilerParams(use_tc_tiling_on_sc=False),
        out_type=jax.ShapeDtypeStruct((2 * n, half_d), jnp.float32),
        scratch_types=[
            pltpu.VMEM_SHARED((n, half_d), jnp.float32),
            pltpu.VMEM_SHARED((n, half_d), jnp.float32),
            pltpu.VMEM((GROUP, CHUNK), jnp.int32),
            pltpu.VMEM((GROUP, CHUNK), jnp.int32),
            pltpu.VMEM((GROUP, CHUNK), jnp.float32),
            pltpu.VMEM((CHUNK, half_d), jnp.float32),
            pltpu.VMEM((CHUNK, half_d), jnp.float32),
            pltpu.VMEM((CHUNK, half_d), jnp.float32),
            pltpu.VMEM((sub_rows, half_d), jnp.float32),
            pltpu.VMEM((sub_rows, half_d), jnp.float32),
            pltpu.VMEM((ZROWS, half_d), jnp.float32),
            pltpu.SemaphoreType.DMA,
            pltpu.SemaphoreType.DMA,
            pltpu.SemaphoreType.DMA,
            pltpu.SemaphoreType.DMA,
            pltpu.SemaphoreType.DMA,
            pltpu.SemaphoreType.DMA,
        ],
    )
    def gcn(table_hbm, src_hbm, dst_hbm, w_hbm, out_hbm,
            ping_sh, pong_sh, src_v, dst_v, w_v, rows0, rows1, rows2, stA, stB, zer,
            gs0, gs1, gs2, ss0, ss1, ss2):
        rows_bufs = (rows0, rows1, rows2)
        gsems = (gs0, gs1, gs2)
        ssems = (ss0, ss1, ss2)
        c = lax.axis_index("c")
        s = lax.axis_index("s")
        r0 = s * rows_per_tile
        hbm0 = c * n

        def zero_body(i, _):
            for j in range(half_d // 16):
                zer[i, pl.ds(j * 16, 16)] = jnp.zeros((16,), jnp.float32)
            return 0
        lax.fori_loop(0, ZROWS, zero_body, 0)

        for sub in range(n_sub):
            rows = pl.ds(r0 + sub * sub_rows, sub_rows)
            hrows = pl.ds(hbm0 + r0 + sub * sub_rows, sub_rows)
            pltpu.sync_copy(table_hbm.at[hrows], stA)
            pltpu.sync_copy(stA, ping_sh.at[rows])
            pltpu.sync_copy(stA, out_hbm.at[hrows])
        for z in range(rows_per_tile // ZROWS):
            pltpu.sync_copy(zer, pong_sh.at[pl.ds(r0 + z * ZROWS, ZROWS)])
        plsc.subcore_barrier()

        def do_layer(gather_sh, acc_sh, is_last):
            def compute(g, rv):
                def edge16(kk, _):
                    wvec = w_v[g, pl.ds(kk * 16, 16)]
                    for e in range(16):
                        i = kk * 16 + e
                        wv = wvec[e]
                        for j in range(half_d // 16):
                            sl = pl.ds(j * 16, 16)
                            rv[i, sl] = rv[i, sl] * wv
                    return 0
                lax.fori_loop(0, CHUNK // 16, edge16, 0)

            def group_body(grp, _):
                g0 = grp * GROUP
                pltpu.sync_copy(src_hbm.at[s, pl.ds(g0, GROUP)], src_v)
                pltpu.sync_copy(dst_hbm.at[s, pl.ds(g0, GROUP)], dst_v)
                pltpu.sync_copy(w_hbm.at[s, pl.ds(g0, GROUP)], w_v)

                gh = [None] * GROUP
                sh = [None] * GROUP
                gh[0] = pltpu.async_copy(gather_sh.at[src_v.at[0]], rows_bufs[0], gsems[0])
                for g in range(GROUP):
                    b = g % 3
                    if g >= 2:
                        sh[g - 2].wait()
                    if g + 1 < GROUP:
                        nb = (g + 1) % 3
                        gh[g + 1] = pltpu.async_copy(
                            gather_sh.at[src_v.at[g + 1]], rows_bufs[nb], gsems[nb])
                    gh[g].wait()
                    compute(g, rows_bufs[b])
                    sh[g] = pltpu.async_copy(
                        rows_bufs[b], acc_sh.at[dst_v.at[g]], ssems[b], add=True)
                sh[GROUP - 2].wait()
                sh[GROUP - 1].wait()
                return 0
            lax.fori_loop(0, n_groups, group_body, 0)
            plsc.subcore_barrier()

            for sub in range(n_sub):
                rows = pl.ds(r0 + sub * sub_rows, sub_rows)
                hrows = pl.ds(hbm0 + r0 + sub * sub_rows, sub_rows)
                pltpu.sync_copy(acc_sh.at[rows], stA)
                pltpu.sync_copy(out_hbm.at[hrows], stB)

                def add_body(i, _):
                    for j in range(half_d // 16):
                        sl = pl.ds(j * 16, 16)
                        v = stB[i, sl] + stA[i, sl]
                        if is_last:
                            v = v * (1.0 / (GCN_LAYERS_K + 1))
                        stB[i, sl] = v
                    return 0
                lax.fori_loop(0, sub_rows, add_body, 0)
                pltpu.sync_copy(stB, out_hbm.at[hrows])

            if not is_last:
                for z in range(rows_per_tile // ZROWS):
                    pltpu.sync_copy(zer, gather_sh.at[pl.ds(r0 + z * ZROWS, ZROWS)])
            plsc.subcore_barrier()

        do_layer(ping_sh, pong_sh, False)
        do_layer(pong_sh, ping_sh, False)
        do_layer(ping_sh, pong_sh, True)

    return gcn(table0, src_r, dst_r, w_r)


def kernel(user_emb, item_emb, edge_index, edge_weight):
    u, d = user_emb.shape
    n = u + item_emb.shape[0]
    e = edge_weight.shape[0]
    half_d = d // 2

    sub_rows = 128
    n_pad = -(-n // (N_SUBCORES * sub_rows)) * (N_SUBCORES * sub_rows)
    rows_per_tile = n_pad // N_SUBCORES

    ept = -(-e // N_SUBCORES)
    ept = -(-ept // (GROUP * CHUNK)) * (GROUP * CHUNK)
    n_chunks = ept // CHUNK
    n_groups = n_chunks // GROUP
    e_pad = ept * N_SUBCORES

    all_emb = jnp.concatenate([user_emb, item_emb], axis=0)
    all_emb = jnp.pad(all_emb, ((0, n_pad - n), (0, 0)))
    table0 = jnp.concatenate([all_emb[:, :half_d], all_emb[:, half_d:]], axis=0)

    src = edge_index[0].astype(jnp.int32)
    dst = edge_index[1].astype(jnp.int32)
    w = edge_weight.astype(jnp.float32)
    pad = e_pad - e
    if pad:
        src = jnp.pad(src, (0, pad))
        dst = jnp.pad(dst, (0, pad))
        w = jnp.pad(w, (0, pad))

    src_r = src.reshape(N_SUBCORES, n_chunks, CHUNK)
    dst_r = dst.reshape(N_SUBCORES, n_chunks, CHUNK)
    w_r = w.reshape(N_SUBCORES, n_chunks, CHUNK)

    out = _gcn_call(table0, src_r, dst_r, w_r, n=n_pad, half_d=half_d,
                    n_groups=n_groups, rows_per_tile=rows_per_tile,
                    sub_rows=sub_rows)
    final = jnp.concatenate([out[:n], out[n_pad:n_pad + n]], axis=1)
    return final[:u], final[u:]


# speedup vs baseline: 6.4525x; 1.7595x over previous
import functools

import jax
import jax.numpy as jnp
from jax import lax
from jax.experimental import pallas as pl
from jax.experimental.pallas import tpu as pltpu
from jax.experimental.pallas import tpu_sc as plsc

N_CORES = 2
N_SUBCORES = 16
N_WORKERS = N_CORES * N_SUBCORES
CHUNK = 128
GROUP = 8
ZROWS = 64
GCN_LAYERS_K = 3


@functools.partial(jax.jit, static_argnames=("n", "half_d", "n_groups", "rows_per_tile", "sub_rows"))
def _gcn_call(table0, src_r, dst_r, w_r, *, n, half_d, n_groups, rows_per_tile, sub_rows):
    n_sub = rows_per_tile // sub_rows
    mesh = plsc.VectorSubcoreMesh(core_axis_name="c", subcore_axis_name="s")

    @functools.partial(
        pl.kernel,
        mesh=mesh,
        compiler_params=pltpu.CompilerParams(use_tc_tiling_on_sc=False),
        out_type=jax.ShapeDtypeStruct((2 * n, half_d), jnp.float32),
        scratch_types=[
            pltpu.VMEM_SHARED((n, half_d), jnp.float32),
            pltpu.VMEM_SHARED((n, half_d), jnp.float32),
            pltpu.VMEM((GROUP, CHUNK), jnp.int32),
            pltpu.VMEM((GROUP, CHUNK), jnp.int32),
            pltpu.VMEM((GROUP, CHUNK), jnp.float32),
            pltpu.VMEM((CHUNK, half_d), jnp.float32),
            pltpu.VMEM((CHUNK, half_d), jnp.float32),
            pltpu.VMEM((CHUNK, half_d), jnp.float32),
            pltpu.VMEM((sub_rows, half_d), jnp.float32),
            pltpu.VMEM((sub_rows, half_d), jnp.float32),
            pltpu.VMEM((ZROWS, half_d), jnp.float32),
            pltpu.SemaphoreType.DMA,
            pltpu.SemaphoreType.DMA,
            pltpu.SemaphoreType.DMA,
            pltpu.SemaphoreType.DMA,
            pltpu.SemaphoreType.DMA,
            pltpu.SemaphoreType.DMA,
        ],
    )
    def gcn(table_hbm, src_hbm, dst_hbm, w_hbm, out_hbm,
            ping_sh, pong_sh, src_v, dst_v, w_v, rows0, rows1, rows2, stA, stB, zer,
            gs0, gs1, gs2, ss0, ss1, ss2):
        rows_bufs = (rows0, rows1, rows2)
        gsems = (gs0, gs1, gs2)
        ssems = (ss0, ss1, ss2)
        c = lax.axis_index("c")
        s = lax.axis_index("s")
        r0 = s * rows_per_tile
        hbm0 = c * n

        def zero_body(i, _):
            for j in range(half_d // 16):
                zer[i, pl.ds(j * 16, 16)] = jnp.zeros((16,), jnp.float32)
            return 0
        lax.fori_loop(0, ZROWS, zero_body, 0)

        for sub in range(n_sub):
            rows = pl.ds(r0 + sub * sub_rows, sub_rows)
            hrows = pl.ds(hbm0 + r0 + sub * sub_rows, sub_rows)
            pltpu.sync_copy(table_hbm.at[hrows], stA)
            pltpu.sync_copy(stA, ping_sh.at[rows])
            pltpu.sync_copy(stA, out_hbm.at[hrows])
        for z in range(rows_per_tile // ZROWS):
            pltpu.sync_copy(zer, pong_sh.at[pl.ds(r0 + z * ZROWS, ZROWS)])
        plsc.subcore_barrier()

        def do_layer(gather_sh, acc_sh, is_last):
            def compute(g, rv):
                @plsc.parallel_loop(0, CHUNK // 16, 1)
                def edge16(kk):
                    wvec = w_v[g, pl.ds(kk * 16, 16)]
                    for e in range(16):
                        i = kk * 16 + e
                        wv = wvec[e]
                        for j in range(half_d // 16):
                            sl = pl.ds(j * 16, 16)
                            rv[i, sl] = rv[i, sl] * wv

            def group_body(grp, _):
                g0 = grp * GROUP
                pltpu.sync_copy(src_hbm.at[s, pl.ds(g0, GROUP)], src_v)
                pltpu.sync_copy(dst_hbm.at[s, pl.ds(g0, GROUP)], dst_v)
                pltpu.sync_copy(w_hbm.at[s, pl.ds(g0, GROUP)], w_v)

                gh = [None] * GROUP
                sh = [None] * GROUP
                gh[0] = pltpu.async_copy(gather_sh.at[src_v.at[0]], rows_bufs[0], gsems[0])
                for g in range(GROUP):
                    b = g % 3
                    if g >= 2:
                        sh[g - 2].wait()
                    if g + 1 < GROUP:
                        nb = (g + 1) % 3
                        gh[g + 1] = pltpu.async_copy(
                            gather_sh.at[src_v.at[g + 1]], rows_bufs[nb], gsems[nb])
                    gh[g].wait()
                    compute(g, rows_bufs[b])
                    sh[g] = pltpu.async_copy(
                        rows_bufs[b], acc_sh.at[dst_v.at[g]], ssems[b], add=True)
                sh[GROUP - 2].wait()
                sh[GROUP - 1].wait()
                return 0
            lax.fori_loop(0, n_groups, group_body, 0)
            plsc.subcore_barrier()

            for sub in range(n_sub):
                rows = pl.ds(r0 + sub * sub_rows, sub_rows)
                hrows = pl.ds(hbm0 + r0 + sub * sub_rows, sub_rows)
                pltpu.sync_copy(acc_sh.at[rows], stA)
                pltpu.sync_copy(out_hbm.at[hrows], stB)

                @plsc.parallel_loop(0, sub_rows, 1)
                def add_body(i):
                    for j in range(half_d // 16):
                        sl = pl.ds(j * 16, 16)
                        v = stB[i, sl] + stA[i, sl]
                        if is_last:
                            v = v * (1.0 / (GCN_LAYERS_K + 1))
                        stB[i, sl] = v
                pltpu.sync_copy(stB, out_hbm.at[hrows])

            if not is_last:
                for z in range(rows_per_tile // ZROWS):
                    pltpu.sync_copy(zer, gather_sh.at[pl.ds(r0 + z * ZROWS, ZROWS)])
            plsc.subcore_barrier()

        do_layer(ping_sh, pong_sh, False)
        do_layer(pong_sh, ping_sh, False)
        do_layer(ping_sh, pong_sh, True)

    return gcn(table0, src_r, dst_r, w_r)


def kernel(user_emb, item_emb, edge_index, edge_weight):
    u, d = user_emb.shape
    n = u + item_emb.shape[0]
    e = edge_weight.shape[0]
    half_d = d // 2

    sub_rows = 128
    n_pad = -(-n // (N_SUBCORES * sub_rows)) * (N_SUBCORES * sub_rows)
    rows_per_tile = n_pad // N_SUBCORES

    ept = -(-e // N_SUBCORES)
    ept = -(-ept // (GROUP * CHUNK)) * (GROUP * CHUNK)
    n_chunks = ept // CHUNK
    n_groups = n_chunks // GROUP
    e_pad = ept * N_SUBCORES

    all_emb = jnp.concatenate([user_emb, item_emb], axis=0)
    all_emb = jnp.pad(all_emb, ((0, n_pad - n), (0, 0)))
    table0 = jnp.concatenate([all_emb[:, :half_d], all_emb[:, half_d:]], axis=0)

    src = edge_index[0].astype(jnp.int32)
    dst = edge_index[1].astype(jnp.int32)
    w = edge_weight.astype(jnp.float32)
    pad = e_pad - e
    if pad:
        src = jnp.pad(src, (0, pad))
        dst = jnp.pad(dst, (0, pad))
        w = jnp.pad(w, (0, pad))

    src_r = src.reshape(N_SUBCORES, n_chunks, CHUNK)
    dst_r = dst.reshape(N_SUBCORES, n_chunks, CHUNK)
    w_r = w.reshape(N_SUBCORES, n_chunks, CHUNK)

    out = _gcn_call(table0, src_r, dst_r, w_r, n=n_pad, half_d=half_d,
                    n_groups=n_groups, rows_per_tile=rows_per_tile,
                    sub_rows=sub_rows)
    final = jnp.concatenate([out[:n], out[n_pad:n_pad + n]], axis=1)
    return final[:u], final[u:]


# speedup vs baseline: 6.6606x; 1.0322x over previous
import functools

import jax
import jax.numpy as jnp
from jax import lax
from jax.experimental import pallas as pl
from jax.experimental.pallas import tpu as pltpu
from jax.experimental.pallas import tpu_sc as plsc

N_CORES = 2
N_SUBCORES = 16
N_WORKERS = N_CORES * N_SUBCORES
CHUNK = 128
GROUP = 8
ZROWS = 64
GCN_LAYERS_K = 3


@functools.partial(jax.jit, static_argnames=("n", "half_d", "n_groups", "rows_per_tile", "sub_rows"))
def _gcn_call(table0, idx_r, w_r, *, n, half_d, n_groups, rows_per_tile, sub_rows):
    n_sub = rows_per_tile // sub_rows
    mesh = plsc.VectorSubcoreMesh(core_axis_name="c", subcore_axis_name="s")

    @functools.partial(
        pl.kernel,
        mesh=mesh,
        compiler_params=pltpu.CompilerParams(use_tc_tiling_on_sc=False),
        out_type=jax.ShapeDtypeStruct((2 * n, half_d), jnp.float32),
        scratch_types=[
            pltpu.VMEM_SHARED((n, half_d), jnp.float32),
            pltpu.VMEM_SHARED((n, half_d), jnp.float32),
            pltpu.VMEM((GROUP, 2, CHUNK), jnp.int32),
            pltpu.VMEM((GROUP, CHUNK), jnp.float32),
            pltpu.VMEM((CHUNK, half_d), jnp.float32),
            pltpu.VMEM((CHUNK, half_d), jnp.float32),
            pltpu.VMEM((CHUNK, half_d), jnp.float32),
            pltpu.VMEM((ZROWS, half_d), jnp.float32),
            pltpu.SemaphoreType.DMA,
            pltpu.SemaphoreType.DMA,
            pltpu.SemaphoreType.DMA,
            pltpu.SemaphoreType.DMA,
            pltpu.SemaphoreType.DMA,
            pltpu.SemaphoreType.DMA,
        ],
    )
    def gcn(table_hbm, idx_hbm, w_hbm, out_hbm,
            ping_sh, pong_sh, idx_v, w_v, rows0, rows1, rows2, zer,
            gs0, gs1, gs2, ss0, ss1, ss2):
        rows_bufs = (rows0, rows1, rows2)
        gsems = (gs0, gs1, gs2)
        ssems = (ss0, ss1, ss2)
        c = lax.axis_index("c")
        s = lax.axis_index("s")
        r0 = s * rows_per_tile
        hbm0 = c * n
        stA, stB = rows0, rows1

        @plsc.parallel_loop(0, ZROWS, 1)
        def zero_body(i):
            for j in range(half_d // 16):
                zer[i, pl.ds(j * 16, 16)] = jnp.zeros((16,), jnp.float32)

        for sub in range(n_sub):
            rows = pl.ds(r0 + sub * sub_rows, sub_rows)
            hrows = pl.ds(hbm0 + r0 + sub * sub_rows, sub_rows)
            pltpu.sync_copy(table_hbm.at[hrows], stA)
            pltpu.sync_copy(stA, ping_sh.at[rows])
            pltpu.sync_copy(stA, out_hbm.at[hrows])
        for z in range(rows_per_tile // ZROWS):
            pltpu.sync_copy(zer, pong_sh.at[pl.ds(r0 + z * ZROWS, ZROWS)])
        plsc.subcore_barrier()

        def do_layer(gather_sh, acc_sh, is_last):
            def compute(g, rv):
                @plsc.parallel_loop(0, CHUNK // 16, 1)
                def edge16(kk):
                    wvec = w_v[g, pl.ds(kk * 16, 16)]
                    for e in range(16):
                        i = kk * 16 + e
                        wv = wvec[e]
                        for j in range(half_d // 16):
                            sl = pl.ds(j * 16, 16)
                            rv[i, sl] = rv[i, sl] * wv

            def group_body(grp, _):
                pltpu.sync_copy(idx_hbm.at[s, pl.ds(grp * GROUP, GROUP)], idx_v)
                pltpu.sync_copy(w_hbm.at[s, pl.ds(grp * GROUP, GROUP)], w_v)

                gh = [None] * GROUP
                sh = [None] * GROUP
                gh[0] = pltpu.async_copy(gather_sh.at[idx_v.at[0, 0]], rows_bufs[0], gsems[0])
                for g in range(GROUP):
                    b = g % 3
                    if g >= 2:
                        sh[g - 2].wait()
                    if g + 1 < GROUP:
                        nb = (g + 1) % 3
                        gh[g + 1] = pltpu.async_copy(
                            gather_sh.at[idx_v.at[g + 1, 0]], rows_bufs[nb], gsems[nb])
                    gh[g].wait()
                    compute(g, rows_bufs[b])
                    sh[g] = pltpu.async_copy(
                        rows_bufs[b], acc_sh.at[idx_v.at[g, 1]], ssems[b], add=True)
                sh[GROUP - 2].wait()
                sh[GROUP - 1].wait()
                return 0
            lax.fori_loop(0, n_groups, group_body, 0)
            plsc.subcore_barrier()

            for sub in range(n_sub):
                rows = pl.ds(r0 + sub * sub_rows, sub_rows)
                hrows = pl.ds(hbm0 + r0 + sub * sub_rows, sub_rows)
                pltpu.sync_copy(acc_sh.at[rows], stA)
                pltpu.sync_copy(out_hbm.at[hrows], stB)

                @plsc.parallel_loop(0, sub_rows, 1)
                def add_body(i):
                    for j in range(half_d // 16):
                        sl = pl.ds(j * 16, 16)
                        v = stB[i, sl] + stA[i, sl]
                        if is_last:
                            v = v * (1.0 / (GCN_LAYERS_K + 1))
                        stB[i, sl] = v
                pltpu.sync_copy(stB, out_hbm.at[hrows])

            if not is_last:
                for z in range(rows_per_tile // ZROWS):
                    pltpu.sync_copy(zer, gather_sh.at[pl.ds(r0 + z * ZROWS, ZROWS)])
            plsc.subcore_barrier()

        do_layer(ping_sh, pong_sh, False)
        do_layer(pong_sh, ping_sh, False)
        do_layer(ping_sh, pong_sh, True)

    return gcn(table0, idx_r, w_r)


def kernel(user_emb, item_emb, edge_index, edge_weight):
    u, d = user_emb.shape
    n = u + item_emb.shape[0]
    e = edge_weight.shape[0]
    half_d = d // 2

    sub_rows = 128
    n_pad = -(-n // (N_SUBCORES * sub_rows)) * (N_SUBCORES * sub_rows)
    rows_per_tile = n_pad // N_SUBCORES

    ept = -(-e // N_SUBCORES)
    ept = -(-ept // (GROUP * CHUNK)) * (GROUP * CHUNK)
    n_chunks = ept // CHUNK
    n_groups = n_chunks // GROUP
    e_pad = ept * N_SUBCORES

    all_emb = jnp.concatenate([user_emb, item_emb], axis=0)
    all_emb = jnp.pad(all_emb, ((0, n_pad - n), (0, 0)))
    table0 = jnp.concatenate([all_emb[:, :half_d], all_emb[:, half_d:]], axis=0)

    src = edge_index[0].astype(jnp.int32)
    dst = edge_index[1].astype(jnp.int32)
    w = edge_weight.astype(jnp.float32)
    pad = e_pad - e
    if pad:
        src = jnp.pad(src, (0, pad))
        dst = jnp.pad(dst, (0, pad))
        w = jnp.pad(w, (0, pad))

    src_r = src.reshape(N_SUBCORES, n_chunks, 1, CHUNK)
    dst_r = dst.reshape(N_SUBCORES, n_chunks, 1, CHUNK)
    idx_r = jnp.concatenate([src_r, dst_r], axis=2)
    w_r = w.reshape(N_SUBCORES, n_chunks, CHUNK)

    out = _gcn_call(table0, idx_r, w_r, n=n_pad, half_d=half_d,
                    n_groups=n_groups, rows_per_tile=rows_per_tile,
                    sub_rows=sub_rows)
    final = jnp.concatenate([out[:n], out[n_pad:n_pad + n]], axis=1)
    return final[:u], final[u:]


# speedup vs baseline: 6.6692x; 1.0013x over previous
import functools

import jax
import jax.numpy as jnp
from jax import lax
from jax.experimental import pallas as pl
from jax.experimental.pallas import tpu as pltpu
from jax.experimental.pallas import tpu_sc as plsc

N_CORES = 2
N_SUBCORES = 16
N_WORKERS = N_CORES * N_SUBCORES
CHUNK = 128
GROUP = 8
ZROWS = 64
GCN_LAYERS_K = 3


@functools.partial(jax.jit, static_argnames=("n", "half_d", "n_groups", "rows_per_tile", "sub_rows"))
def _gcn_call(table0, idx_r, w_r, *, n, half_d, n_groups, rows_per_tile, sub_rows):
    n_sub = rows_per_tile // sub_rows
    mesh = plsc.VectorSubcoreMesh(core_axis_name="c", subcore_axis_name="s")

    @functools.partial(
        pl.kernel,
        mesh=mesh,
        compiler_params=pltpu.CompilerParams(use_tc_tiling_on_sc=False),
        out_type=jax.ShapeDtypeStruct((2 * n, half_d), jnp.float32),
        scratch_types=[
            pltpu.VMEM_SHARED((n, half_d), jnp.float32),
            pltpu.VMEM_SHARED((n, half_d), jnp.float32),
            pltpu.VMEM((2 * GROUP, CHUNK), jnp.int32),
            pltpu.VMEM((GROUP, CHUNK), jnp.float32),
            pltpu.VMEM((CHUNK, half_d), jnp.float32),
            pltpu.VMEM((CHUNK, half_d), jnp.float32),
            pltpu.VMEM((CHUNK, half_d), jnp.float32),
            pltpu.VMEM((ZROWS, half_d), jnp.float32),
            pltpu.SemaphoreType.DMA,
            pltpu.SemaphoreType.DMA,
            pltpu.SemaphoreType.DMA,
            pltpu.SemaphoreType.DMA,
            pltpu.SemaphoreType.DMA,
            pltpu.SemaphoreType.DMA,
        ],
    )
    def gcn(table_hbm, idx_hbm, w_hbm, out_hbm,
            ping_sh, pong_sh, idx_v, w_v, rows0, rows1, rows2, zer,
            gs0, gs1, gs2, ss0, ss1, ss2):
        rows_bufs = (rows0, rows1, rows2)
        gsems = (gs0, gs1, gs2)
        ssems = (ss0, ss1, ss2)
        c = lax.axis_index("c")
        s = lax.axis_index("s")
        r0 = s * rows_per_tile
        hbm0 = c * n
        stA, stB = rows0, rows1

        @plsc.parallel_loop(0, ZROWS, 1)
        def zero_body(i):
            for j in range(half_d // 16):
                zer[i, pl.ds(j * 16, 16)] = jnp.zeros((16,), jnp.float32)

        for sub in range(n_sub):
            rows = pl.ds(r0 + sub * sub_rows, sub_rows)
            hrows = pl.ds(hbm0 + r0 + sub * sub_rows, sub_rows)
            pltpu.sync_copy(table_hbm.at[hrows], stA)
            pltpu.sync_copy(stA, ping_sh.at[rows])
            pltpu.sync_copy(stA, out_hbm.at[hrows])
        for z in range(rows_per_tile // ZROWS):
            pltpu.sync_copy(zer, pong_sh.at[pl.ds(r0 + z * ZROWS, ZROWS)])
        plsc.subcore_barrier()

        def do_layer(gather_sh, acc_sh, is_last):
            def compute(g, rv):
                @plsc.parallel_loop(0, CHUNK // 16, 1)
                def edge16(kk):
                    wvec = w_v[g, pl.ds(kk * 16, 16)]
                    for e in range(16):
                        i = kk * 16 + e
                        wv = wvec[e]
                        for j in range(half_d // 16):
                            sl = pl.ds(j * 16, 16)
                            rv[i, sl] = rv[i, sl] * wv

            def group_body(grp, _):
                pltpu.sync_copy(idx_hbm.at[s, pl.ds(grp * 2 * GROUP, 2 * GROUP)], idx_v)
                pltpu.sync_copy(w_hbm.at[s, pl.ds(grp * GROUP, GROUP)], w_v)

                gh = [None] * GROUP
                sh = [None] * GROUP
                gh[0] = pltpu.async_copy(gather_sh.at[idx_v.at[0]], rows_bufs[0], gsems[0])
                for g in range(GROUP):
                    b = g % 3
                    if g >= 2:
                        sh[g - 2].wait()
                    if g + 1 < GROUP:
                        nb = (g + 1) % 3
                        gh[g + 1] = pltpu.async_copy(
                            gather_sh.at[idx_v.at[2 * (g + 1)]], rows_bufs[nb], gsems[nb])
                    gh[g].wait()
                    compute(g, rows_bufs[b])
                    sh[g] = pltpu.async_copy(
                        rows_bufs[b], acc_sh.at[idx_v.at[2 * g + 1]], ssems[b], add=True)
                sh[GROUP - 2].wait()
                sh[GROUP - 1].wait()
                return 0
            lax.fori_loop(0, n_groups, group_body, 0)
            plsc.subcore_barrier()

            for sub in range(n_sub):
                rows = pl.ds(r0 + sub * sub_rows, sub_rows)
                hrows = pl.ds(hbm0 + r0 + sub * sub_rows, sub_rows)
                pltpu.sync_copy(acc_sh.at[rows], stA)
                pltpu.sync_copy(out_hbm.at[hrows], stB)

                @plsc.parallel_loop(0, sub_rows, 1)
                def add_body(i):
                    for j in range(half_d // 16):
                        sl = pl.ds(j * 16, 16)
                        v = stB[i, sl] + stA[i, sl]
                        if is_last:
                            v = v * (1.0 / (GCN_LAYERS_K + 1))
                        stB[i, sl] = v
                pltpu.sync_copy(stB, out_hbm.at[hrows])

            if not is_last:
                for z in range(rows_per_tile // ZROWS):
                    pltpu.sync_copy(zer, gather_sh.at[pl.ds(r0 + z * ZROWS, ZROWS)])
            plsc.subcore_barrier()

        do_layer(ping_sh, pong_sh, False)
        do_layer(pong_sh, ping_sh, False)
        do_layer(ping_sh, pong_sh, True)

    return gcn(table0, idx_r, w_r)


def kernel(user_emb, item_emb, edge_index, edge_weight):
    u, d = user_emb.shape
    n = u + item_emb.shape[0]
    e = edge_weight.shape[0]
    half_d = d // 2

    sub_rows = 128
    n_pad = -(-n // (N_SUBCORES * sub_rows)) * (N_SUBCORES * sub_rows)
    rows_per_tile = n_pad // N_SUBCORES

    ept = -(-e // N_SUBCORES)
    ept = -(-ept // (GROUP * CHUNK)) * (GROUP * CHUNK)
    n_chunks = ept // CHUNK
    n_groups = n_chunks // GROUP
    e_pad = ept * N_SUBCORES

    all_emb = jnp.concatenate([user_emb, item_emb], axis=0)
    all_emb = jnp.pad(all_emb, ((0, n_pad - n), (0, 0)))
    table0 = jnp.concatenate([all_emb[:, :half_d], all_emb[:, half_d:]], axis=0)

    src = edge_index[0].astype(jnp.int32)
    dst = edge_index[1].astype(jnp.int32)
    w = edge_weight.astype(jnp.float32)
    pad = e_pad - e
    if pad:
        src = jnp.pad(src, (0, pad))
        dst = jnp.pad(dst, (0, pad))
        w = jnp.pad(w, (0, pad))

    src_r = src.reshape(N_SUBCORES, n_chunks, 1, CHUNK)
    dst_r = dst.reshape(N_SUBCORES, n_chunks, 1, CHUNK)
    idx_r = jnp.concatenate([src_r, dst_r], axis=2).reshape(N_SUBCORES, 2 * n_chunks, CHUNK)
    w_r = w.reshape(N_SUBCORES, n_chunks, CHUNK)

    out = _gcn_call(table0, idx_r, w_r, n=n_pad, half_d=half_d,
                    n_groups=n_groups, rows_per_tile=rows_per_tile,
                    sub_rows=sub_rows)
    final = jnp.concatenate([out[:n], out[n_pad:n_pad + n]], axis=1)
    return final[:u], final[u:]
